# Initial kernel scaffold; baseline (speedup 1.0000x reference)
import jax
import jax.numpy as jnp
from jax.experimental import pallas as pl


def kernel(hidden, codebook, W_in, W_out, ln_g, ln_b, active_mask):
    raise NotImplementedError("write your pallas kernel here")


# speedup vs baseline: 1.6016x; 1.6016x over previous
import functools

import jax
import jax.numpy as jnp
from jax import lax
from jax.experimental import pallas as pl
from jax.experimental.pallas import tpu as pltpu
from jax.experimental.pallas import tpu_sc as plsc

_BT = 512


def _s1_body(h_ref, w_ref, cb_ref, idx_ref, acc_ref):
    i = pl.program_id(0)
    h = h_ref[...]
    cb = cb_ref[...]
    z = lax.dot_general(h, w_ref[...], (((1,), (1,)), ((), ())),
                        preferred_element_type=jnp.float32)
    s = lax.dot_general(z, cb, (((1,), (1,)), ((), ())),
                        preferred_element_type=jnp.float32)
    ones = jnp.ones((1, cb.shape[1]), jnp.float32)
    cnorm = lax.dot_general(ones, cb * cb, (((1,), (1,)), ((), ())),
                            preferred_element_type=jnp.float32)
    d = cnorm - 2.0 * s
    dmin = jnp.min(d, axis=1, keepdims=True)
    cols = lax.broadcasted_iota(jnp.int32, d.shape, 1)
    idx_ref[...] = jnp.min(jnp.where(d <= dmin, cols, d.shape[1]),
                           axis=1, keepdims=True)
    znorm = jnp.sum(z * z, axis=1, keepdims=True)
    part = jnp.sum(znorm + dmin, axis=(0, 1), keepdims=True)

    @pl.when(i == 0)
    def _init():
        acc_ref[...] = part

    @pl.when(i != 0)
    def _accum():
        acc_ref[...] += part


def _s3_body(zq_ref, h_ref, a_ref, w_ref, g_ref, b_ref, o_ref):
    out = lax.dot_general(zq_ref[...], w_ref[...], (((1,), (1,)), ((), ())),
                          preferred_element_type=jnp.float32)
    x = h_ref[...] + a_ref[...] * out
    mu = jnp.mean(x, axis=1, keepdims=True)
    xc = x - mu
    var = jnp.mean(xc * xc, axis=1, keepdims=True)
    o_ref[...] = xc * lax.rsqrt(var + 1e-5) * g_ref[...] + b_ref[...]


def _make_sc_gather(num_tokens, dim):
    info = plsc.get_sparse_core_info()
    nc, ns = info.num_cores, info.num_subcores
    nw = nc * ns
    b_per_w = num_tokens // nw
    ch = 64
    n_ch = b_per_w // ch
    mesh = plsc.VectorSubcoreMesh(core_axis_name="c", subcore_axis_name="s")

    @functools.partial(
        pl.kernel, mesh=mesh,
        out_type=jax.ShapeDtypeStruct((num_tokens, dim), jnp.float32),
        scratch_types=[
            pltpu.VMEM((ch,), jnp.int32),
            pltpu.VMEM((ch, dim), jnp.float32),
            pltpu.SemaphoreType.DMA,
        ],
    )
    def gather(table_hbm, idx_hbm, out_hbm, idx_v, rows_v, sem):
        wid = lax.axis_index("s") * nc + lax.axis_index("c")
        base = wid * b_per_w
        for c in range(n_ch):
            off = base + c * ch
            pltpu.sync_copy(idx_hbm.at[pl.ds(off, ch)], idx_v)
            pltpu.async_copy(table_hbm.at[idx_v], rows_v, sem).wait()
            pltpu.sync_copy(rows_v, out_hbm.at[pl.ds(off, ch)])

    return gather


def kernel(hidden, codebook, W_in, W_out, ln_g, ln_b, active_mask):
    d = hidden.shape[-1]
    n = hidden.shape[0] * hidden.shape[1]
    c = codebook.shape[0]
    h2 = hidden.reshape(n, d)
    nblk = n // _BT

    idx, acc = pl.pallas_call(
        _s1_body,
        grid=(nblk,),
        in_specs=[
            pl.BlockSpec((_BT, d), lambda i: (i, 0)),
            pl.BlockSpec((d, d), lambda i: (0, 0)),
            pl.BlockSpec((c, d), lambda i: (0, 0)),
        ],
        out_specs=[
            pl.BlockSpec((_BT, 1), lambda i: (i, 0)),
            pl.BlockSpec((1, 1), lambda i: (0, 0)),
        ],
        out_shape=[
            jax.ShapeDtypeStruct((n, 1), jnp.int32),
            jax.ShapeDtypeStruct((1, 1), jnp.float32),
        ],
    )(h2, W_in, codebook)

    z_q = _make_sc_gather(n, d)(codebook, idx.reshape(n))

    active_f = active_mask.reshape(n, 1).astype(jnp.float32)
    h_comm = pl.pallas_call(
        _s3_body,
        grid=(nblk,),
        in_specs=[
            pl.BlockSpec((_BT, d), lambda i: (i, 0)),
            pl.BlockSpec((_BT, d), lambda i: (i, 0)),
            pl.BlockSpec((_BT, 1), lambda i: (i, 0)),
            pl.BlockSpec((d, d), lambda i: (0, 0)),
            pl.BlockSpec((1, d), lambda i: (0, 0)),
            pl.BlockSpec((1, d), lambda i: (0, 0)),
        ],
        out_specs=pl.BlockSpec((_BT, d), lambda i: (i, 0)),
        out_shape=jax.ShapeDtypeStruct((n, d), jnp.float32),
    )(z_q, h2, active_f, W_out, ln_g.reshape(1, d), ln_b.reshape(1, d))

    vq_loss = (1.0 + 0.25) * acc[0, 0] / (n * d)
    return h_comm.reshape(hidden.shape), vq_loss
